# baseline (speedup 1.0000x reference)
import jax
import jax.numpy as jnp
from jax import lax
from jax.experimental import pallas as pl
from jax.experimental.pallas import tpu as pltpu
from jax.experimental.pallas import tpu_sc as plsc

_SEQ = 8192
_DIM = 1024
_NC = 2
_NS = 16
_NW = _NC * _NS
_ROWS_PER_W = _SEQ // _NW


def _copy_body(table_hbm, out_hbm):
    wid = lax.axis_index("s") * _NC + lax.axis_index("c")
    base = wid * _ROWS_PER_W
    pltpu.sync_copy(
        table_hbm.at[pl.ds(base, _ROWS_PER_W)],
        out_hbm.at[pl.ds(base, _ROWS_PER_W)],
    )


def kernel(x, emb_weight):
    del x
    mesh = plsc.VectorSubcoreMesh(core_axis_name="c", subcore_axis_name="s")
    run = pl.kernel(
        _copy_body,
        mesh=mesh,
        out_type=jax.ShapeDtypeStruct((_SEQ, _DIM), jnp.float32),
    )
    return run(emb_weight)


# speedup vs baseline: 24.4585x; 24.4585x over previous
import jax
import jax.numpy as jnp
from jax import lax
from jax.experimental import pallas as pl
from jax.experimental.pallas import tpu as pltpu
from jax.experimental.pallas import tpu_sc as plsc

_SEQ = 8192
_DIM = 1024
_NC = 2
_NS = 16
_NW = _NC * _NS
_ROWS_PER_W = _SEQ // _NW
_NBUF = 4
_CH = 16
_NCH = _ROWS_PER_W // _CH


def _copy_body(table_hbm, out_hbm, *scratch):
    bufs = scratch[:_NBUF]
    isems = scratch[_NBUF:2 * _NBUF]
    osems = scratch[2 * _NBUF:]
    wid = lax.axis_index("s") * _NC + lax.axis_index("c")
    base = wid * _ROWS_PER_W

    def in_copy(c):
        b = c % _NBUF
        return pltpu.make_async_copy(
            table_hbm.at[pl.ds(base + c * _CH, _CH)], bufs[b], isems[b])

    def out_copy(c):
        b = c % _NBUF
        return pltpu.make_async_copy(
            bufs[b], out_hbm.at[pl.ds(base + c * _CH, _CH)], osems[b])

    for c in range(_NBUF):
        in_copy(c).start()
    for c in range(_NCH):
        in_copy(c).wait()
        out_copy(c).start()
        if c + _NBUF < _NCH:
            out_copy(c).wait()
            in_copy(c + _NBUF).start()
    for c in range(_NCH - _NBUF, _NCH):
        out_copy(c).wait()


def kernel(x, emb_weight):
    del x
    mesh = plsc.VectorSubcoreMesh(core_axis_name="c", subcore_axis_name="s")
    run = pl.kernel(
        _copy_body,
        mesh=mesh,
        out_type=jax.ShapeDtypeStruct((_SEQ, _DIM), jnp.float32),
        scratch_types=(
            [pltpu.VMEM((_CH, _DIM), jnp.float32) for _ in range(_NBUF)]
            + [pltpu.SemaphoreType.DMA for _ in range(2 * _NBUF)]
        ),
    )
    return run(emb_weight)


# speedup vs baseline: 24.9925x; 1.0218x over previous
import jax
import jax.numpy as jnp
from jax import lax
from jax.experimental import pallas as pl
from jax.experimental.pallas import tpu as pltpu
from jax.experimental.pallas import tpu_sc as plsc

_SEQ = 8192
_DIM = 1024
_NC = 2
_NS = 16
_NW = _NC * _NS
_ROWS_PER_W = _SEQ // _NW
_NBUF = 3
_CH = 32
_NCH = _ROWS_PER_W // _CH


def _copy_body(table_hbm, out_hbm, *scratch):
    bufs = scratch[:_NBUF]
    isems = scratch[_NBUF:2 * _NBUF]
    osems = scratch[2 * _NBUF:]
    wid = lax.axis_index("s") * _NC + lax.axis_index("c")
    base = wid * _ROWS_PER_W

    def in_copy(c):
        b = c % _NBUF
        return pltpu.make_async_copy(
            table_hbm.at[pl.ds(base + c * _CH, _CH)], bufs[b], isems[b])

    def out_copy(c):
        b = c % _NBUF
        return pltpu.make_async_copy(
            bufs[b], out_hbm.at[pl.ds(base + c * _CH, _CH)], osems[b])

    for c in range(_NBUF):
        in_copy(c).start()
    for c in range(_NCH):
        in_copy(c).wait()
        out_copy(c).start()
        if c + _NBUF < _NCH:
            out_copy(c).wait()
            in_copy(c + _NBUF).start()
    for c in range(_NCH - _NBUF, _NCH):
        out_copy(c).wait()


def kernel(x, emb_weight):
    del x
    mesh = plsc.VectorSubcoreMesh(core_axis_name="c", subcore_axis_name="s")
    run = pl.kernel(
        _copy_body,
        mesh=mesh,
        out_type=jax.ShapeDtypeStruct((_SEQ, _DIM), jnp.float32),
        scratch_types=(
            [pltpu.VMEM((_CH, _DIM), jnp.float32) for _ in range(_NBUF)]
            + [pltpu.SemaphoreType.DMA for _ in range(2 * _NBUF)]
        ),
    )
    return run(emb_weight)


# speedup vs baseline: 42.1979x; 1.6884x over previous
import jax
import jax.numpy as jnp
from jax.experimental import pallas as pl
from jax.experimental.pallas import tpu as pltpu

_SEQ = 8192
_DIM = 1024
_BLK = 512


def _copy_block(in_ref, out_ref):
    out_ref[...] = in_ref[...]


def kernel(x, emb_weight):
    del x
    return pl.pallas_call(
        _copy_block,
        grid=(_SEQ // _BLK,),
        in_specs=[pl.BlockSpec((_BLK, _DIM), lambda i: (i, 0))],
        out_specs=pl.BlockSpec((_BLK, _DIM), lambda i: (i, 0)),
        out_shape=jax.ShapeDtypeStruct((_SEQ, _DIM), jnp.float32),
    )(emb_weight)
